# baseline (speedup 1.0000x reference)
import functools

import jax
import jax.numpy as jnp
from jax import lax
from jax.experimental import pallas as pl
from jax.experimental.pallas import tpu as pltpu
from jax.experimental.pallas import tpu_sc as plsc

VOCAB_SIZE = 50257
DIM = 1024
CONTEXT_LENGTH = 2048
BATCH = 4
SEQ_LEN = 2048

_NC = 2
_NS = 16
_NW = _NC * _NS
_BS = BATCH * SEQ_LEN
_SW = SEQ_LEN // _NW
_C = 16
_SCHUNKS = _SW // _C
_NG = _SCHUNKS * BATCH
_NBUF = 5
_PBUF = 2
_LANES = 16
_GROUPS = DIM // _LANES


def _make_kernel():
  mesh = plsc.VectorSubcoreMesh(core_axis_name="c", subcore_axis_name="s")

  @functools.partial(
      pl.kernel,
      out_type=jax.ShapeDtypeStruct((_BS, DIM), jnp.float32),
      mesh=mesh,
      scratch_types=[
          pltpu.VMEM((BATCH * _SW,), jnp.int32),
          pltpu.VMEM((_NBUF, _C, DIM), jnp.float32),
          pltpu.VMEM((_PBUF, _C, DIM), jnp.float32),
          pltpu.SemaphoreType.DMA((_NBUF,)),
          pltpu.SemaphoreType.DMA((_PBUF,)),
          pltpu.SemaphoreType.DMA((_NBUF,)),
          pltpu.SemaphoreType.DMA((BATCH,)),
      ],
  )
  def k(idx_hbm, tok_hbm, pos_hbm, out_hbm,
        idx_all, tok_v, pos_v, gsem, psem, osem, isem):
    wid = lax.axis_index("s") * _NC + lax.axis_index("c")
    s0 = wid * _SW

    for b in range(BATCH):
      pltpu.async_copy(idx_hbm.at[b, pl.ds(s0, _SW)],
                       idx_all.at[pl.ds(b * _SW, _SW)], isem.at[b])
    for b in range(BATCH):
      pltpu.make_async_copy(idx_hbm.at[b, pl.ds(s0, _SW)],
                            idx_all.at[pl.ds(b * _SW, _SW)], isem.at[b]).wait()

    def start_gather(g):
      sc = lax.div(g, BATCH) if not isinstance(g, int) else g // BATCH
      b = lax.rem(g, BATCH) if not isinstance(g, int) else g % BATCH
      slot = lax.rem(g, _NBUF) if not isinstance(g, int) else g % _NBUF
      pltpu.async_copy(
          tok_hbm.at[idx_all.at[pl.ds(b * _SW + sc * _C, _C)]],
          tok_v.at[slot], gsem.at[slot])

    def start_pos(sc):
      pb = lax.rem(sc, _PBUF) if not isinstance(sc, int) else sc % _PBUF
      pltpu.async_copy(pos_hbm.at[pl.ds(s0 + sc * _C, _C)],
                       pos_v.at[pb], psem.at[pb])

    def wait_store(slot):
      pltpu.make_async_copy(
          tok_v.at[slot], out_hbm.at[pl.ds(0, _C)], osem.at[slot]).wait()

    start_pos(0)
    start_pos(1)
    for g in range(_NBUF):
      start_gather(g)

    @pl.loop(0, _SCHUNKS)
    def sc_body(sc):
      pb = lax.rem(sc, _PBUF)
      slots = [lax.rem(sc * BATCH + b, _NBUF) for b in range(BATCH)]

      for t in slots:
        pltpu.make_async_copy(
            tok_hbm.at[idx_all.at[pl.ds(0, _C)]], tok_v.at[t], gsem.at[t]
        ).wait()
      pltpu.make_async_copy(
          pos_hbm.at[pl.ds(0, _C)], pos_v.at[pb], psem.at[pb]).wait()

      @pl.loop(0, _C)
      def add_row(i):
        for j in range(_GROUPS):
          sl = pl.ds(j * _LANES, _LANES)
          p = pos_v[pb, i, sl]
          for t in slots:
            plsc.addupdate(tok_v.at[t, i, sl], p)

      @pl.when(sc + 2 < _SCHUNKS)
      def _():
        start_pos(sc + 2)

      for b in range(BATCH):
        off = b * SEQ_LEN + s0 + sc * _C
        pltpu.async_copy(tok_v.at[slots[b]], out_hbm.at[pl.ds(off, _C)],
                         osem.at[slots[b]])

      for b in range(BATCH):
        g = sc * BATCH + _NBUF + b

        @pl.when(g < _NG)
        def _():
          wait_store(lax.rem(g, _NBUF))
          start_gather(g)

    for t in range(_NBUF):
      wait_store(t)

  return k


_kernel_fn = _make_kernel()


def kernel(in_idx, tok_emb, pos_emb):
  out = _kernel_fn(in_idx.astype(jnp.int32), tok_emb, pos_emb)
  return out.reshape(BATCH, SEQ_LEN, DIM)


# speedup vs baseline: 1.0718x; 1.0718x over previous
import functools

import jax
import jax.numpy as jnp
from jax import lax
from jax.experimental import pallas as pl
from jax.experimental.pallas import tpu as pltpu
from jax.experimental.pallas import tpu_sc as plsc

VOCAB_SIZE = 50257
DIM = 1024
CONTEXT_LENGTH = 2048
BATCH = 4
SEQ_LEN = 2048

_NC = 2
_NS = 16
_NW = _NC * _NS
_BS = BATCH * SEQ_LEN
_SW = SEQ_LEN // _NW
_C = 16
_SCHUNKS = _SW // _C
_NG = _SCHUNKS * BATCH
_NBUF = 5
_PBUF = 2
_LANES = 16
_GROUPS = DIM // _LANES


def _make_kernel():
  mesh = plsc.VectorSubcoreMesh(core_axis_name="c", subcore_axis_name="s")

  @functools.partial(
      pl.kernel,
      out_type=jax.ShapeDtypeStruct((_BS, DIM), jnp.float32),
      mesh=mesh,
      scratch_types=[
          pltpu.VMEM((BATCH * _SW,), jnp.int32),
          pltpu.VMEM((_NBUF, _C, DIM), jnp.float32),
          pltpu.VMEM((_PBUF, _C, DIM), jnp.float32),
          pltpu.SemaphoreType.DMA((_NBUF,)),
          pltpu.SemaphoreType.DMA((_PBUF,)),
          pltpu.SemaphoreType.DMA((_NBUF,)),
          pltpu.SemaphoreType.DMA((BATCH,)),
      ],
  )
  def k(idx_hbm, tok_hbm, pos_hbm, out_hbm,
        idx_all, tok_v, pos_v, gsem, psem, osem, isem):
    wid = lax.axis_index("s") * _NC + lax.axis_index("c")
    s0 = wid * _SW

    for b in range(BATCH):
      pltpu.async_copy(idx_hbm.at[b, pl.ds(s0, _SW)],
                       idx_all.at[pl.ds(b * _SW, _SW)], isem.at[b])
    for b in range(BATCH):
      pltpu.make_async_copy(idx_hbm.at[b, pl.ds(s0, _SW)],
                            idx_all.at[pl.ds(b * _SW, _SW)], isem.at[b]).wait()

    def start_gather(g):
      sc, b = divmod(g, BATCH)
      pltpu.async_copy(
          tok_hbm.at[idx_all.at[pl.ds(b * _SW + sc * _C, _C)]],
          tok_v.at[g % _NBUF], gsem.at[g % _NBUF])

    def start_pos(sc):
      pltpu.async_copy(pos_hbm.at[pl.ds(s0 + sc * _C, _C)],
                       pos_v.at[sc % _PBUF], psem.at[sc % _PBUF])

    def wait_store(slot):
      pltpu.make_async_copy(
          tok_v.at[slot], out_hbm.at[pl.ds(0, _C)], osem.at[slot]).wait()

    start_pos(0)
    start_pos(1)
    for g in range(_NBUF):
      start_gather(g)

    for sc in range(_SCHUNKS):
      slots = [(sc * BATCH + b) % _NBUF for b in range(BATCH)]

      for t in slots:
        pltpu.make_async_copy(
            tok_hbm.at[idx_all.at[pl.ds(0, _C)]], tok_v.at[t], gsem.at[t]
        ).wait()
      pltpu.make_async_copy(
          pos_hbm.at[pl.ds(0, _C)], pos_v.at[sc % _PBUF],
          psem.at[sc % _PBUF]).wait()

      @pl.loop(0, _C)
      def add_row(i):
        for j in range(0, _GROUPS, 2):
          sl0 = pl.ds(j * _LANES, _LANES)
          sl1 = pl.ds((j + 1) * _LANES, _LANES)
          p0 = pos_v[sc % _PBUF, i, sl0]
          p1 = pos_v[sc % _PBUF, i, sl1]
          for t in slots:
            plsc.addupdate(tok_v.at[t, i, sl0], p0)
          for t in slots:
            plsc.addupdate(tok_v.at[t, i, sl1], p1)

      if sc + 2 < _SCHUNKS:
        start_pos(sc + 2)

      for b in range(BATCH):
        t = slots[b]
        off = b * SEQ_LEN + s0 + sc * _C
        pltpu.async_copy(tok_v.at[t], out_hbm.at[pl.ds(off, _C)], osem.at[t])

      for g in range(sc * BATCH + _NBUF, min((sc + 1) * BATCH + _NBUF, _NG)):
        wait_store(g % _NBUF)
        start_gather(g)

    for t in range(_NBUF):
      wait_store(t)

  return k


_kernel_fn = _make_kernel()


def kernel(in_idx, tok_emb, pos_emb):
  out = _kernel_fn(in_idx.astype(jnp.int32), tok_emb, pos_emb)
  return out.reshape(BATCH, SEQ_LEN, DIM)


# speedup vs baseline: 1.0972x; 1.0237x over previous
import functools

import jax
import jax.numpy as jnp
from jax import lax
from jax.experimental import pallas as pl
from jax.experimental.pallas import tpu as pltpu
from jax.experimental.pallas import tpu_sc as plsc

VOCAB_SIZE = 50257
DIM = 1024
CONTEXT_LENGTH = 2048
BATCH = 4
SEQ_LEN = 2048

_NC = 2
_NS = 16
_NW = _NC * _NS
_BS = BATCH * SEQ_LEN
_SW = SEQ_LEN // _NW
_C = 16
_SCHUNKS = _SW // _C
_NG = _SCHUNKS * BATCH
_NBUF = 5
_PBUF = 2
_LANES = 16
_GROUPS = DIM // _LANES


def _make_kernel():
  mesh = plsc.VectorSubcoreMesh(core_axis_name="c", subcore_axis_name="s")

  @functools.partial(
      pl.kernel,
      out_type=jax.ShapeDtypeStruct((_BS, DIM), jnp.float32),
      mesh=mesh,
      scratch_types=[
          pltpu.VMEM((BATCH * _SW,), jnp.int32),
          pltpu.VMEM((_NBUF, _C, DIM), jnp.float32),
          pltpu.VMEM((_PBUF, _C, DIM), jnp.float32),
          pltpu.SemaphoreType.DMA((_NBUF,)),
          pltpu.SemaphoreType.DMA((_PBUF,)),
          pltpu.SemaphoreType.DMA((_NBUF,)),
          pltpu.SemaphoreType.DMA((BATCH,)),
      ],
  )
  def k(idx_hbm, tok_hbm, pos_hbm, out_hbm,
        idx_all, tok_v, pos_v, gsem, psem, osem, isem):
    wid = lax.axis_index("s") * _NC + lax.axis_index("c")
    s0 = wid * _SW

    for b in range(BATCH):
      pltpu.async_copy(idx_hbm.at[b, pl.ds(s0, _SW)],
                       idx_all.at[pl.ds(b * _SW, _SW)], isem.at[b])
    for b in range(BATCH):
      pltpu.make_async_copy(idx_hbm.at[b, pl.ds(s0, _SW)],
                            idx_all.at[pl.ds(b * _SW, _SW)], isem.at[b]).wait()

    def start_gather(g):
      sc, b = divmod(g, BATCH)
      pltpu.async_copy(
          tok_hbm.at[idx_all.at[pl.ds(b * _SW + sc * _C, _C)]],
          tok_v.at[g % _NBUF], gsem.at[g % _NBUF])

    def start_pos(sc):
      pltpu.async_copy(pos_hbm.at[pl.ds(s0 + sc * _C, _C)],
                       pos_v.at[sc % _PBUF], psem.at[sc % _PBUF])

    def wait_store(slot):
      pltpu.make_async_copy(
          tok_v.at[slot], out_hbm.at[pl.ds(0, _C)], osem.at[slot]).wait()

    start_pos(0)
    start_pos(1)
    for g in range(_NBUF):
      start_gather(g)

    for sc in range(_SCHUNKS):
      slots = [(sc * BATCH + b) % _NBUF for b in range(BATCH)]

      for t in slots:
        pltpu.make_async_copy(
            tok_hbm.at[idx_all.at[pl.ds(0, _C)]], tok_v.at[t], gsem.at[t]
        ).wait()
      pltpu.make_async_copy(
          pos_hbm.at[pl.ds(0, _C)], pos_v.at[sc % _PBUF],
          psem.at[sc % _PBUF]).wait()

      @pl.loop(0, _C)
      def add_row(i):
        for j in range(0, _GROUPS, 4):
          sls = [pl.ds((j + u) * _LANES, _LANES) for u in range(4)]
          ps = [pos_v[sc % _PBUF, i, s] for s in sls]
          for u in range(4):
            for t in slots:
              plsc.addupdate(tok_v.at[t, i, sls[u]], ps[u])

      if sc + 2 < _SCHUNKS:
        start_pos(sc + 2)

      for b in range(BATCH):
        t = slots[b]
        off = b * SEQ_LEN + s0 + sc * _C
        pltpu.async_copy(tok_v.at[t], out_hbm.at[pl.ds(off, _C)], osem.at[t])

      for g in range(sc * BATCH + _NBUF, min((sc + 1) * BATCH + _NBUF, _NG)):
        wait_store(g % _NBUF)
        start_gather(g)

    for t in range(_NBUF):
      wait_store(t)

  return k


_kernel_fn = _make_kernel()


def kernel(in_idx, tok_emb, pos_emb):
  out = _kernel_fn(in_idx.astype(jnp.int32), tok_emb, pos_emb)
  return out.reshape(BATCH, SEQ_LEN, DIM)


# speedup vs baseline: 1.1039x; 1.0061x over previous
import functools

import jax
import jax.numpy as jnp
from jax import lax
from jax.experimental import pallas as pl
from jax.experimental.pallas import tpu as pltpu
from jax.experimental.pallas import tpu_sc as plsc

VOCAB_SIZE = 50257
DIM = 1024
CONTEXT_LENGTH = 2048
BATCH = 4
SEQ_LEN = 2048

_NC = 2
_NS = 16
_NW = _NC * _NS
_BS = BATCH * SEQ_LEN
_SW = SEQ_LEN // _NW
_C = 16
_SCHUNKS = _SW // _C
_NG = _SCHUNKS * BATCH
_NBUF = 5
_PBUF = 2
_LANES = 16
_GROUPS = DIM // _LANES


def _make_kernel():
  mesh = plsc.VectorSubcoreMesh(core_axis_name="c", subcore_axis_name="s")

  @functools.partial(
      pl.kernel,
      out_type=jax.ShapeDtypeStruct((_BS, DIM), jnp.float32),
      mesh=mesh,
      scratch_types=[
          pltpu.VMEM((BATCH * _SW,), jnp.int32),
          pltpu.VMEM((_NBUF, _C, DIM), jnp.float32),
          pltpu.VMEM((_PBUF, _C, DIM), jnp.float32),
          pltpu.SemaphoreType.DMA((_NBUF,)),
          pltpu.SemaphoreType.DMA((_PBUF,)),
          pltpu.SemaphoreType.DMA((_NBUF,)),
          pltpu.SemaphoreType.DMA((BATCH,)),
      ],
  )
  def k(idx_hbm, tok_hbm, pos_hbm, out_hbm,
        idx_all, tok_v, pos_v, gsem, psem, osem, isem):
    wid = lax.axis_index("s") * _NC + lax.axis_index("c")
    s0 = wid * _SW

    for b in range(BATCH):
      pltpu.async_copy(idx_hbm.at[b, pl.ds(s0, _SW)],
                       idx_all.at[pl.ds(b * _SW, _SW)], isem.at[b])
    for b in range(BATCH):
      pltpu.make_async_copy(idx_hbm.at[b, pl.ds(s0, _SW)],
                            idx_all.at[pl.ds(b * _SW, _SW)], isem.at[b]).wait()

    def start_gather(g):
      sc, b = divmod(g, BATCH)
      pltpu.async_copy(
          tok_hbm.at[idx_all.at[pl.ds(b * _SW + sc * _C, _C)]],
          tok_v.at[g % _NBUF], gsem.at[g % _NBUF])

    def start_pos(sc):
      pltpu.async_copy(pos_hbm.at[pl.ds(s0 + sc * _C, _C)],
                       pos_v.at[sc % _PBUF], psem.at[sc % _PBUF])

    def wait_store(slot):
      pltpu.make_async_copy(
          tok_v.at[slot], out_hbm.at[pl.ds(0, _C)], osem.at[slot]).wait()

    start_pos(0)
    start_pos(1)
    for g in range(_NBUF):
      start_gather(g)

    for sc in range(_SCHUNKS):
      slots = [(sc * BATCH + b) % _NBUF for b in range(BATCH)]

      for t in slots:
        pltpu.make_async_copy(
            tok_hbm.at[idx_all.at[pl.ds(0, _C)]], tok_v.at[t], gsem.at[t]
        ).wait()
      pltpu.make_async_copy(
          pos_hbm.at[pl.ds(0, _C)], pos_v.at[sc % _PBUF],
          psem.at[sc % _PBUF]).wait()

      @pl.loop(0, _C)
      def add_row(i):
        for j in range(0, _GROUPS, 8):
          sls = [pl.ds((j + u) * _LANES, _LANES) for u in range(8)]
          ps = [pos_v[sc % _PBUF, i, s] for s in sls]
          for u in range(8):
            for t in slots:
              plsc.addupdate(tok_v.at[t, i, sls[u]], ps[u])

      if sc + 2 < _SCHUNKS:
        start_pos(sc + 2)

      for b in range(BATCH):
        t = slots[b]
        off = b * SEQ_LEN + s0 + sc * _C
        pltpu.async_copy(tok_v.at[t], out_hbm.at[pl.ds(off, _C)], osem.at[t])

      for g in range(sc * BATCH + _NBUF, min((sc + 1) * BATCH + _NBUF, _NG)):
        wait_store(g % _NBUF)
        start_gather(g)

    for t in range(_NBUF):
      wait_store(t)

  return k


_kernel_fn = _make_kernel()


def kernel(in_idx, tok_emb, pos_emb):
  out = _kernel_fn(in_idx.astype(jnp.int32), tok_emb, pos_emb)
  return out.reshape(BATCH, SEQ_LEN, DIM)
